# baseline (speedup 1.0000x reference)
import functools

import jax
import jax.numpy as jnp
from jax import lax
from jax.experimental import pallas as pl
from jax.experimental.pallas import tpu as pltpu
from jax.experimental.pallas import tpu_sc as plsc

N = 10000
E = 320000
D = 128
G = 128

NC = 2
NS = 16
NW = NC * NS
EPT = E // NW
CHUNK = 128
CHP = 80
EPT_P = CHP * CHUNK
PH = 4
CPP = CHP // PH
WIN = CPP + 4
CHG = PH * CPP + WIN
RPT = 632
NPAD = NS * RPT
DUMMY = N + 8

_MESH = plsc.VectorSubcoreMesh(core_axis_name="c", subcore_axis_name="s",
                               num_cores=NC, num_subcores=NS)


@functools.partial(
    pl.kernel,
    out_type=jax.ShapeDtypeStruct((NC, NPAD, 16), jnp.float32),
    mesh=_MESH,
    scratch_types=[
        pltpu.VMEM((CHP, CHUNK), jnp.int32),
        pltpu.VMEM((CHUNK, 16), jnp.float32),
        pltpu.VMEM_SHARED((NPAD, 16), jnp.float32),
    ],
)
def _sc_degree(col_hbm, ones_hbm, zeros_hbm, deg_out, col_v, ones_v, acc):
    c = lax.axis_index("c")
    s = lax.axis_index("s")
    w = c * NS + s
    pltpu.sync_copy(col_hbm.at[w], col_v)
    pltpu.sync_copy(ones_hbm, ones_v)
    pltpu.sync_copy(zeros_hbm, acc.at[pl.ds(s * RPT, RPT)])
    plsc.subcore_barrier()

    def body(j, carry):
        pltpu.sync_copy(ones_v, acc.at[col_v.at[j]], add=True)
        return carry

    lax.fori_loop(0, CHP, body, 0)
    plsc.subcore_barrier()
    pltpu.sync_copy(acc.at[pl.ds(s * RPT, RPT)],
                    deg_out.at[c, pl.ds(s * RPT, RPT)])


@functools.partial(
    pl.kernel,
    out_type=jax.ShapeDtypeStruct((NC, NPAD, D), jnp.float32),
    mesh=_MESH,
    scratch_types=[
        pltpu.VMEM((WIN, CHUNK), jnp.int32),
        pltpu.VMEM((WIN, CHUNK), jnp.int32),
        pltpu.VMEM((WIN, CHUNK), jnp.int32),
        pltpu.VMEM((WIN, CHUNK), jnp.int32),
        pltpu.VMEM((2, CHUNK, D), jnp.float32),
        pltpu.SemaphoreType.DMA,
        pltpu.SemaphoreType.DMA,
        pltpu.VMEM_SHARED((NPAD, D), jnp.float32),
    ],
)
def _sc_scatter(hs_hbm, rowp_hbm, colp_hbm, zeros_hbm, out_hbm,
                row_a, row_b, col_a, col_b, rows_v, sem0, sem1, acc):
    c = lax.axis_index("c")
    s = lax.axis_index("s")
    w = c * NS + s
    pltpu.sync_copy(zeros_hbm, acc.at[pl.ds(s * RPT, RPT)])
    plsc.subcore_barrier()

    buf0 = rows_v.at[0]
    buf1 = rows_v.at[1]
    for p in range(PH):
        rv = row_a if p % 2 == 0 else row_b
        cv = col_a if p % 2 == 0 else col_b
        pltpu.sync_copy(rowp_hbm.at[w * PH + p], rv)
        pltpu.sync_copy(colp_hbm.at[w * PH + p], cv)
        if p == 0:
            pltpu.async_copy(hs_hbm.at[rv.at[0]], buf0, sem0)
            pltpu.async_copy(hs_hbm.at[rv.at[1]], buf1, sem1)

        def body(jj, carry, rv=rv, cv=cv):
            j0 = 2 * jj
            j1 = 2 * jj + 1
            pltpu.make_async_copy(hs_hbm.at[rv.at[j0]], buf0, sem0).wait()
            pltpu.sync_copy(buf0, acc.at[cv.at[j0]], add=True)
            pltpu.async_copy(hs_hbm.at[rv.at[j0 + 2]], buf0, sem0)
            pltpu.make_async_copy(hs_hbm.at[rv.at[j1]], buf1, sem1).wait()
            pltpu.sync_copy(buf1, acc.at[cv.at[j1]], add=True)
            pltpu.async_copy(hs_hbm.at[rv.at[j1 + 2]], buf1, sem1)
            return carry

        lax.fori_loop(0, CPP // 2, body, 0)

    lv = row_a if (PH - 1) % 2 == 0 else row_b
    pltpu.make_async_copy(hs_hbm.at[lv.at[CPP]], buf0, sem0).wait()
    pltpu.make_async_copy(hs_hbm.at[lv.at[CPP + 1]], buf1, sem1).wait()
    plsc.subcore_barrier()
    pltpu.sync_copy(acc.at[pl.ds(s * RPT, RPT)],
                    out_hbm.at[c, pl.ds(s * RPT, RPT)])


BN = 1000
NB = N // BN


def _tc_scale_matmul_body(dega_ref, degb_ref, x_ref, w_ref,
                          hs_ref, dinv_ref):
    deg = 1.0 + dega_ref[...] + degb_ref[...]
    dinv = lax.rsqrt(deg)
    h = jnp.dot(x_ref[...], w_ref[...], preferred_element_type=jnp.float32)
    hs_ref[...] = h * dinv
    dinv_ref[...] = dinv


def _tc_layer_body(pa_ref, pb_ref, hs_ref, dinv_ref, w_ref, b_ref,
                   hs2_ref):
    dinv = dinv_ref[...]
    conv = dinv * (pa_ref[...] + pb_ref[...] + hs_ref[...]) + b_ref[...]
    a = jnp.maximum(conv, 0.0)
    hs2_ref[...] = dinv * jnp.dot(a, w_ref[...],
                                  preferred_element_type=jnp.float32)


def _tc_pool_body(pa_ref, pb_ref, hs_ref, dinv_ref, b_ref, batch_ref,
                  out_ref, acc_ref, cnt_ref):
    i = pl.program_id(0)
    dinv = dinv_ref[...]
    conv = dinv * (pa_ref[...] + pb_ref[...] + hs_ref[...]) + b_ref[...]
    h2 = jnp.maximum(conv, 0.0)
    gids = lax.broadcasted_iota(jnp.int32, (1, G), 1)
    onehot = (batch_ref[...] == gids).astype(jnp.float32)
    sums = lax.dot_general(onehot, h2, (((0,), (0,)), ((), ())),
                           preferred_element_type=jnp.float32)
    cnt = jnp.sum(onehot, axis=0).reshape(G, 1)

    @pl.when(i == 0)
    def _():
        acc_ref[...] = jnp.zeros_like(acc_ref)
        cnt_ref[...] = jnp.zeros_like(cnt_ref)

    acc_ref[...] += sums
    cnt_ref[...] += cnt

    @pl.when(i == NB - 1)
    def _():
        out_ref[...] = acc_ref[...] / jnp.maximum(cnt_ref[...], 1.0)


_row_spec = pl.BlockSpec((BN, D), lambda i: (i, 0))
_col_spec = pl.BlockSpec((BN, 1), lambda i: (i, 0))
_w_spec = pl.BlockSpec((D, D), lambda i: (0, 0))
_b_spec = pl.BlockSpec((1, D), lambda i: (0, 0))

_tc_scale_matmul = pl.pallas_call(
    _tc_scale_matmul_body,
    grid=(NB,),
    in_specs=[_col_spec, _col_spec, _row_spec, _w_spec],
    out_specs=[_row_spec, _col_spec],
    out_shape=[jax.ShapeDtypeStruct((N, D), jnp.float32),
               jax.ShapeDtypeStruct((N, 1), jnp.float32)],
)

_tc_layer = pl.pallas_call(
    _tc_layer_body,
    grid=(NB,),
    in_specs=[_row_spec, _row_spec, _row_spec, _col_spec, _w_spec, _b_spec],
    out_specs=_row_spec,
    out_shape=jax.ShapeDtypeStruct((N, D), jnp.float32),
)

_tc_pool = pl.pallas_call(
    _tc_pool_body,
    grid=(NB,),
    in_specs=[_row_spec, _row_spec, _row_spec, _col_spec, _b_spec,
              pl.BlockSpec((BN, 1), lambda i: (i, 0))],
    out_specs=pl.BlockSpec((G, D), lambda i: (0, 0)),
    out_shape=jax.ShapeDtypeStruct((G, D), jnp.float32),
    scratch_shapes=[pltpu.VMEM((G, D), jnp.float32),
                    pltpu.VMEM((G, 1), jnp.float32)],
)


def kernel(x, edge_index, batch, W1, b1, W2, b2):
    row = edge_index[0].astype(jnp.int32)
    col = edge_index[1].astype(jnp.int32)

    gpad = CHG * CHUNK - EPT
    row_t = jnp.pad(row.reshape(NW, EPT), ((0, 0), (0, gpad))) \
        .reshape(NW, CHG, CHUNK)
    row_ph = jnp.stack([row_t[:, p * CPP: p * CPP + WIN]
                        for p in range(PH)], axis=1) \
        .reshape(NW * PH, WIN, CHUNK)
    col_t = jnp.pad(col.reshape(NW, EPT), ((0, 0), (0, EPT_P - EPT)),
                    constant_values=DUMMY).reshape(NW, CHP, CHUNK)
    col_tp = jnp.pad(col_t, ((0, 0), (0, WIN), (0, 0)),
                     constant_values=DUMMY)
    col_ph = jnp.stack([col_tp[:, p * CPP: p * CPP + WIN]
                        for p in range(PH)], axis=1) \
        .reshape(NW * PH, WIN, CHUNK)

    ones16 = jnp.ones((CHUNK, 16), jnp.float32)
    zeros16 = jnp.zeros((RPT, 16), jnp.float32)
    zerosD = jnp.zeros((RPT, D), jnp.float32)

    deg_parts = _sc_degree(col_t, ones16, zeros16)
    dega = deg_parts[0, :N, 0:1]
    degb = deg_parts[1, :N, 0:1]

    hs1, dinv = _tc_scale_matmul(dega, degb, x, W1)

    p1 = _sc_scatter(hs1, row_ph, col_ph, zerosD)
    hs2 = _tc_layer(p1[0, :N], p1[1, :N], hs1, dinv, W2,
                    b1.reshape(1, D))

    p2 = _sc_scatter(hs2, row_ph, col_ph, zerosD)
    out = _tc_pool(p2[0, :N], p2[1, :N], hs2, dinv, b2.reshape(1, D),
                   batch.astype(jnp.int32).reshape(N, 1))
    return out


# speedup vs baseline: 1.4942x; 1.4942x over previous
import functools

import jax
import jax.numpy as jnp
from jax import lax
from jax.experimental import pallas as pl
from jax.experimental.pallas import tpu as pltpu
from jax.experimental.pallas import tpu_sc as plsc

N = 10000
E = 320000
D = 128
G = 128

NC = 2
NS = 16
NW = NC * NS
EPT = E // NW
CHUNK = 128
CHP = 80
EPT_P = CHP * CHUNK
PH = 4
CPP = CHP // PH
WIN = CPP + 4
CHG = PH * CPP + WIN
RPT = 632
NPAD = NS * RPT
DUMMY = N + 8

_MESH = plsc.VectorSubcoreMesh(core_axis_name="c", subcore_axis_name="s",
                               num_cores=NC, num_subcores=NS)


@functools.partial(
    pl.kernel,
    out_type=jax.ShapeDtypeStruct((NC, NPAD, 16), jnp.float32),
    mesh=_MESH,
    scratch_types=[
        pltpu.VMEM((CHP, CHUNK), jnp.int32),
        pltpu.VMEM((CHUNK, 16), jnp.float32),
        pltpu.VMEM_SHARED((NPAD, 16), jnp.float32),
    ],
)
def _sc_degree(col_hbm, ones_hbm, zeros_hbm, deg_out, col_v, ones_v, acc):
    c = lax.axis_index("c")
    s = lax.axis_index("s")
    w = c * NS + s
    pltpu.sync_copy(col_hbm.at[w], col_v)
    pltpu.sync_copy(ones_hbm, ones_v)
    pltpu.sync_copy(zeros_hbm, acc.at[pl.ds(s * RPT, RPT)])
    plsc.subcore_barrier()

    def body(j, carry):
        pltpu.sync_copy(ones_v, acc.at[col_v.at[j]], add=True)
        return carry

    lax.fori_loop(0, CHP, body, 0)
    plsc.subcore_barrier()
    pltpu.sync_copy(acc.at[pl.ds(s * RPT, RPT)],
                    deg_out.at[c, pl.ds(s * RPT, RPT)])


@functools.partial(
    pl.kernel,
    out_type=jax.ShapeDtypeStruct((NC, NPAD, D), jnp.float32),
    mesh=_MESH,
    scratch_types=[
        pltpu.VMEM((WIN, CHUNK), jnp.int32),
        pltpu.VMEM((WIN, CHUNK), jnp.int32),
        pltpu.VMEM((WIN, CHUNK), jnp.int32),
        pltpu.VMEM((WIN, CHUNK), jnp.int32),
        pltpu.VMEM((2, CHUNK, D), jnp.float32),
        pltpu.SemaphoreType.DMA,
        pltpu.SemaphoreType.DMA,
        pltpu.VMEM_SHARED((NPAD, D), jnp.float32),
    ],
)
def _sc_scatter(hs_hbm, rowp_hbm, colp_hbm, zeros_hbm, out_hbm,
                row_a, row_b, col_a, col_b, rows_v, sem0, sem1, acc):
    c = lax.axis_index("c")
    s = lax.axis_index("s")
    w = c * NS + s
    pltpu.sync_copy(zeros_hbm, acc.at[pl.ds(s * RPT, RPT)])
    plsc.subcore_barrier()

    buf0 = rows_v.at[0]
    buf1 = rows_v.at[1]
    for p in range(PH):
        rv = row_a if p % 2 == 0 else row_b
        cv = col_a if p % 2 == 0 else col_b
        pltpu.sync_copy(rowp_hbm.at[w * PH + p], rv)
        pltpu.sync_copy(colp_hbm.at[w * PH + p], cv)
        def body(jj, carry, rv=rv, cv=cv):
            j0 = 2 * jj
            j1 = 2 * jj + 1
            pltpu.async_copy(hs_hbm.at[rv.at[j0]], buf0, sem0).wait()
            pltpu.sync_copy(buf0, acc.at[cv.at[j0]], add=True)
            pltpu.async_copy(hs_hbm.at[rv.at[j1]], buf1, sem1).wait()
            pltpu.sync_copy(buf1, acc.at[cv.at[j1]], add=True)
            return carry

        lax.fori_loop(0, CPP // 2, body, 0)
    plsc.subcore_barrier()
    pltpu.sync_copy(acc.at[pl.ds(s * RPT, RPT)],
                    out_hbm.at[c, pl.ds(s * RPT, RPT)])


BN = 1000
NB = N // BN


def _tc_scale_matmul_body(dega_ref, degb_ref, x_ref, w_ref,
                          hs_ref, dinv_ref):
    deg = 1.0 + dega_ref[0] + degb_ref[0]
    dinv = lax.rsqrt(deg)
    h = jnp.dot(x_ref[...], w_ref[...], preferred_element_type=jnp.float32)
    hs_ref[...] = h * dinv
    dinv_ref[...] = dinv


def _tc_layer_body(pa_ref, pb_ref, hs_ref, dinv_ref, w_ref, b_ref,
                   hs2_ref):
    dinv = dinv_ref[...]
    conv = dinv * (pa_ref[0] + pb_ref[0] + hs_ref[...]) + b_ref[...]
    a = jnp.maximum(conv, 0.0)
    hs2_ref[...] = dinv * jnp.dot(a, w_ref[...],
                                  preferred_element_type=jnp.float32)


def _tc_pool_body(pa_ref, pb_ref, hs_ref, dinv_ref, b_ref, batch_ref,
                  out_ref, acc_ref, cnt_ref):
    i = pl.program_id(0)
    dinv = dinv_ref[...]
    conv = dinv * (pa_ref[0] + pb_ref[0] + hs_ref[...]) + b_ref[...]
    h2 = jnp.maximum(conv, 0.0)
    gids = lax.broadcasted_iota(jnp.int32, (1, G), 1)
    onehot = (batch_ref[...] == gids).astype(jnp.float32)
    sums = lax.dot_general(onehot, h2, (((0,), (0,)), ((), ())),
                           preferred_element_type=jnp.float32)
    cnt = jnp.sum(onehot, axis=0).reshape(G, 1)

    @pl.when(i == 0)
    def _():
        acc_ref[...] = jnp.zeros_like(acc_ref)
        cnt_ref[...] = jnp.zeros_like(cnt_ref)

    acc_ref[...] += sums
    cnt_ref[...] += cnt

    @pl.when(i == NB - 1)
    def _():
        out_ref[...] = acc_ref[...] / jnp.maximum(cnt_ref[...], 1.0)


_row_spec = pl.BlockSpec((BN, D), lambda i: (i, 0))
_col_spec = pl.BlockSpec((BN, 1), lambda i: (i, 0))
_pa_spec = pl.BlockSpec((1, BN, D), lambda i: (0, i, 0))
_pb_spec = pl.BlockSpec((1, BN, D), lambda i: (1, i, 0))
_da_spec = pl.BlockSpec((1, BN, 1), lambda i: (0, i, 0))
_db_spec = pl.BlockSpec((1, BN, 1), lambda i: (1, i, 0))
_w_spec = pl.BlockSpec((D, D), lambda i: (0, 0))
_b_spec = pl.BlockSpec((1, D), lambda i: (0, 0))

_tc_scale_matmul = pl.pallas_call(
    _tc_scale_matmul_body,
    grid=(NB,),
    in_specs=[_da_spec, _db_spec, _row_spec, _w_spec],
    out_specs=[_row_spec, _col_spec],
    out_shape=[jax.ShapeDtypeStruct((N, D), jnp.float32),
               jax.ShapeDtypeStruct((N, 1), jnp.float32)],
)

_tc_layer = pl.pallas_call(
    _tc_layer_body,
    grid=(NB,),
    in_specs=[_pa_spec, _pb_spec, _row_spec, _col_spec, _w_spec, _b_spec],
    out_specs=_row_spec,
    out_shape=jax.ShapeDtypeStruct((N, D), jnp.float32),
)

_tc_pool = pl.pallas_call(
    _tc_pool_body,
    grid=(NB,),
    in_specs=[_pa_spec, _pb_spec, _row_spec, _col_spec, _b_spec,
              pl.BlockSpec((BN, 1), lambda i: (i, 0))],
    out_specs=pl.BlockSpec((G, D), lambda i: (0, 0)),
    out_shape=jax.ShapeDtypeStruct((G, D), jnp.float32),
    scratch_shapes=[pltpu.VMEM((G, D), jnp.float32),
                    pltpu.VMEM((G, 1), jnp.float32)],
)


def kernel(x, edge_index, batch, W1, b1, W2, b2):
    row = edge_index[0].astype(jnp.int32)
    col = edge_index[1].astype(jnp.int32)

    gpad = CHG * CHUNK - EPT
    row_t = jnp.pad(row.reshape(NW, EPT), ((0, 0), (0, gpad))) \
        .reshape(NW, CHG, CHUNK)
    row_ph = jnp.stack([row_t[:, p * CPP: p * CPP + WIN]
                        for p in range(PH)], axis=1) \
        .reshape(NW * PH, WIN, CHUNK)
    col_t = jnp.pad(col.reshape(NW, EPT), ((0, 0), (0, EPT_P - EPT)),
                    constant_values=DUMMY).reshape(NW, CHP, CHUNK)
    col_tp = jnp.pad(col_t, ((0, 0), (0, WIN), (0, 0)),
                     constant_values=DUMMY)
    col_ph = jnp.stack([col_tp[:, p * CPP: p * CPP + WIN]
                        for p in range(PH)], axis=1) \
        .reshape(NW * PH, WIN, CHUNK)

    ones16 = jnp.ones((CHUNK, 16), jnp.float32)
    zeros16 = jnp.zeros((RPT, 16), jnp.float32)
    zerosD = jnp.zeros((RPT, D), jnp.float32)

    deg_parts = _sc_degree(col_t, ones16, zeros16)

    degs = deg_parts[:, :, 0:1]
    hs1, dinv = _tc_scale_matmul(degs, degs, x, W1)

    p1 = _sc_scatter(hs1, row_ph, col_ph, zerosD)
    hs2 = _tc_layer(p1, p1, hs1, dinv, W2, b1.reshape(1, D))

    p2 = _sc_scatter(hs2, row_ph, col_ph, zerosD)
    out = _tc_pool(p2, p2, hs2, dinv, b2.reshape(1, D),
                   batch.astype(jnp.int32).reshape(N, 1))
    return out


# speedup vs baseline: 2.1748x; 1.4555x over previous
import functools

import jax
import jax.numpy as jnp
from jax import lax
from jax.experimental import pallas as pl
from jax.experimental.pallas import tpu as pltpu
from jax.experimental.pallas import tpu_sc as plsc

N = 10000
E = 320000
D = 128
G = 128

NC = 2
NS = 16
NW = NC * NS
EPT = E // NW
CHUNK = 128
CHP = -(-EPT // CHUNK)
EPT_P = CHP * CHUNK
RPT = 632
NPAD = NS * RPT
DUMMY = N + 8

_MESH = plsc.VectorSubcoreMesh(core_axis_name="c", subcore_axis_name="s",
                               num_cores=NC, num_subcores=NS)


@functools.partial(
    pl.kernel,
    out_type=jax.ShapeDtypeStruct((NC, NPAD, 16), jnp.float32),
    mesh=_MESH,
    scratch_types=[
        pltpu.VMEM((CHP, CHUNK), jnp.int32),
        pltpu.VMEM((CHUNK, 16), jnp.float32),
        pltpu.VMEM_SHARED((NPAD, 16), jnp.float32),
    ],
)
def _sc_degree(col_hbm, ones_hbm, zeros_hbm, deg_out, col_v, ones_v, acc):
    c = lax.axis_index("c")
    s = lax.axis_index("s")
    w = c * NS + s
    pltpu.sync_copy(col_hbm.at[w], col_v)
    pltpu.sync_copy(ones_hbm, ones_v)
    pltpu.sync_copy(zeros_hbm, acc.at[pl.ds(s * RPT, RPT)])
    plsc.subcore_barrier()

    def body(j, carry):
        pltpu.sync_copy(ones_v, acc.at[col_v.at[j]], add=True)
        return carry

    lax.fori_loop(0, CHP, body, 0)
    plsc.subcore_barrier()
    pltpu.sync_copy(acc.at[pl.ds(s * RPT, RPT)],
                    deg_out.at[c, pl.ds(s * RPT, RPT)])


@functools.partial(
    pl.kernel,
    out_type=jax.ShapeDtypeStruct((NC, NPAD, D), jnp.float32),
    mesh=_MESH,
    scratch_types=[
        pltpu.VMEM((CHP, CHUNK), jnp.int32),
        pltpu.VMEM((CHP, CHUNK), jnp.int32),
        pltpu.VMEM((CHUNK, D), jnp.float32),
        pltpu.SemaphoreType.DMA,
        pltpu.VMEM_SHARED((NPAD, D), jnp.float32),
    ],
)
def _sc_scatter(hs_hbm, row_hbm, col_hbm, zeros_hbm, out_hbm,
                row_v, col_v, rows_v, sem, acc):
    c = lax.axis_index("c")
    s = lax.axis_index("s")
    w = c * NS + s
    pltpu.sync_copy(row_hbm.at[w], row_v)
    pltpu.sync_copy(col_hbm.at[w], col_v)
    pltpu.sync_copy(zeros_hbm, acc.at[pl.ds(s * RPT, RPT)])
    plsc.subcore_barrier()

    def body(j, carry):
        pltpu.async_copy(hs_hbm.at[row_v.at[j]], rows_v, sem).wait()
        pltpu.sync_copy(rows_v, acc.at[col_v.at[j]], add=True)
        return carry

    lax.fori_loop(0, CHP, body, 0)
    plsc.subcore_barrier()
    pltpu.sync_copy(acc.at[pl.ds(s * RPT, RPT)],
                    out_hbm.at[c, pl.ds(s * RPT, RPT)])


BN = 1000
NB = N // BN


def _tc_scale_matmul_body(dega_ref, degb_ref, x_ref, w_ref,
                          hs_ref, dinv_ref):
    deg = 1.0 + dega_ref[0] + degb_ref[0]
    dinv = lax.rsqrt(deg)
    h = jnp.dot(x_ref[...], w_ref[...], preferred_element_type=jnp.float32)
    hs_ref[...] = h * dinv
    dinv_ref[...] = dinv


def _tc_layer_body(pa_ref, pb_ref, hs_ref, dinv_ref, w_ref, b_ref,
                   hs2_ref):
    dinv = dinv_ref[...]
    conv = dinv * (pa_ref[0] + pb_ref[0] + hs_ref[...]) + b_ref[...]
    a = jnp.maximum(conv, 0.0)
    hs2_ref[...] = dinv * jnp.dot(a, w_ref[...],
                                  preferred_element_type=jnp.float32)


def _tc_pool_body(pa_ref, pb_ref, hs_ref, dinv_ref, b_ref, batch_ref,
                  out_ref, acc_ref, cnt_ref):
    i = pl.program_id(0)
    dinv = dinv_ref[...]
    conv = dinv * (pa_ref[0] + pb_ref[0] + hs_ref[...]) + b_ref[...]
    h2 = jnp.maximum(conv, 0.0)
    gids = lax.broadcasted_iota(jnp.int32, (1, G), 1)
    onehot = (batch_ref[...] == gids).astype(jnp.float32)
    sums = lax.dot_general(onehot, h2, (((0,), (0,)), ((), ())),
                           preferred_element_type=jnp.float32)
    cnt = jnp.sum(onehot, axis=0).reshape(G, 1)

    @pl.when(i == 0)
    def _():
        acc_ref[...] = jnp.zeros_like(acc_ref)
        cnt_ref[...] = jnp.zeros_like(cnt_ref)

    acc_ref[...] += sums
    cnt_ref[...] += cnt

    @pl.when(i == NB - 1)
    def _():
        out_ref[...] = acc_ref[...] / jnp.maximum(cnt_ref[...], 1.0)


_row_spec = pl.BlockSpec((BN, D), lambda i: (i, 0))
_col_spec = pl.BlockSpec((BN, 1), lambda i: (i, 0))
_pa_spec = pl.BlockSpec((1, BN, D), lambda i: (0, i, 0))
_pb_spec = pl.BlockSpec((1, BN, D), lambda i: (1, i, 0))
_da_spec = pl.BlockSpec((1, BN, 1), lambda i: (0, i, 0))
_db_spec = pl.BlockSpec((1, BN, 1), lambda i: (1, i, 0))
_w_spec = pl.BlockSpec((D, D), lambda i: (0, 0))
_b_spec = pl.BlockSpec((1, D), lambda i: (0, 0))

_tc_scale_matmul = pl.pallas_call(
    _tc_scale_matmul_body,
    grid=(NB,),
    in_specs=[_da_spec, _db_spec, _row_spec, _w_spec],
    out_specs=[_row_spec, _col_spec],
    out_shape=[jax.ShapeDtypeStruct((N, D), jnp.float32),
               jax.ShapeDtypeStruct((N, 1), jnp.float32)],
)

_tc_layer = pl.pallas_call(
    _tc_layer_body,
    grid=(NB,),
    in_specs=[_pa_spec, _pb_spec, _row_spec, _col_spec, _w_spec, _b_spec],
    out_specs=_row_spec,
    out_shape=jax.ShapeDtypeStruct((N, D), jnp.float32),
)

_tc_pool = pl.pallas_call(
    _tc_pool_body,
    grid=(NB,),
    in_specs=[_pa_spec, _pb_spec, _row_spec, _col_spec, _b_spec,
              pl.BlockSpec((BN, 1), lambda i: (i, 0))],
    out_specs=pl.BlockSpec((G, D), lambda i: (0, 0)),
    out_shape=jax.ShapeDtypeStruct((G, D), jnp.float32),
    scratch_shapes=[pltpu.VMEM((G, D), jnp.float32),
                    pltpu.VMEM((G, 1), jnp.float32)],
)


def kernel(x, edge_index, batch, W1, b1, W2, b2):
    row = edge_index[0].astype(jnp.int32)
    col = edge_index[1].astype(jnp.int32)

    pad = EPT_P - EPT
    row_t = jnp.pad(row.reshape(NW, EPT), ((0, 0), (0, pad))) \
        .reshape(NW, CHP, CHUNK)
    col_t = jnp.pad(col.reshape(NW, EPT), ((0, 0), (0, pad)),
                    constant_values=DUMMY).reshape(NW, CHP, CHUNK)

    ones16 = jnp.ones((CHUNK, 16), jnp.float32)
    zeros16 = jnp.zeros((RPT, 16), jnp.float32)
    zerosD = jnp.zeros((RPT, D), jnp.float32)

    deg_parts = _sc_degree(col_t, ones16, zeros16)

    degs = deg_parts[:, :, 0:1]
    hs1, dinv = _tc_scale_matmul(degs, degs, x, W1)

    p1 = _sc_scatter(hs1, row_t, col_t, zerosD)
    hs2 = _tc_layer(p1, p1, hs1, dinv, W2, b1.reshape(1, D))

    p2 = _sc_scatter(hs2, row_t, col_t, zerosD)
    out = _tc_pool(p2, p2, hs2, dinv, b2.reshape(1, D),
                   batch.astype(jnp.int32).reshape(N, 1))
    return out
